# baseline (speedup 1.0000x reference)
import functools

import jax
import jax.numpy as jnp
from jax import lax
from jax.experimental import pallas as pl
from jax.experimental.pallas import tpu as pltpu
from jax.experimental.pallas import tpu_sc as plsc

_B, _S, _D = 4, 4096, 1024
_NC, _NS, _L = 2, 16, 16
_NW = _NC * _NS
_SPW = _S // _NW
_CS = 16
_NCHUNK = _SPW // _CS
_CE = _CS * _D
_NBUF = 5
_NITEMS = _NCHUNK * _B
_PD = 3


def _make_sc_kernel():
    mesh = plsc.VectorSubcoreMesh(core_axis_name="c", subcore_axis_name="s")

    @functools.partial(
        pl.kernel,
        out_type=jax.ShapeDtypeStruct((_B * _S * _D,), jnp.float32),
        mesh=mesh,
        scratch_types=[pltpu.VMEM((_CE,), jnp.float32)] * (_NBUF + 2)
        + [pltpu.SemaphoreType.DMA] * (2 * _NBUF + 2),
    )
    def sc_add(x_hbm, pos_hbm, out_hbm, *scratch):
        x_bufs = scratch[:_NBUF]
        pos_bufs = scratch[_NBUF:_NBUF + 2]
        sems = scratch[_NBUF + 2:]
        in_sems = sems[:_NBUF]
        out_sems = sems[_NBUF:2 * _NBUF]
        pos_sems = sems[2 * _NBUF:]

        wid = lax.axis_index("s") * _NC + lax.axis_index("c")
        base = wid * (_SPW * _D)

        def x_off(k):
            c, b = divmod(k, _B)
            return b * (_S * _D) + base + c * _CE

        def gather_x(k):
            j = k % _NBUF
            return pltpu.async_copy(
                x_hbm.at[pl.ds(x_off(k), _CE)], x_bufs[j], in_sems[j]
            )

        def gather_pos(c):
            return pltpu.async_copy(
                pos_hbm.at[pl.ds(base + c * _CE, _CE)],
                pos_bufs[c % 2],
                pos_sems[c % 2],
            )

        pos_d = {0: gather_pos(0), 1: gather_pos(1)}
        x_d = {k: gather_x(k) for k in range(_PD)}
        scat_d = {}

        for k in range(_NITEMS):
            j = k % _NBUF
            c = k // _B
            if k % _B == 0:
                if 1 <= c and c + 1 < _NCHUNK:
                    pos_d[c + 1] = gather_pos(c + 1)
                pos_d[c].wait()
            x_d[k].wait()

            @plsc.parallel_loop(0, _CE // _L, 1, unroll=8)
            def _add(i, j=j, pb=c % 2):
                plsc.addupdate(
                    x_bufs[j].at[pl.ds(i * _L, _L)],
                    pos_bufs[pb][pl.ds(i * _L, _L)],
                )

            scat_d[k] = pltpu.async_copy(
                x_bufs[j], out_hbm.at[pl.ds(x_off(k), _CE)], out_sems[j]
            )
            kn = k + _PD
            if kn < _NITEMS:
                if kn - _NBUF >= 0:
                    scat_d[kn - _NBUF].wait()
                x_d[kn] = gather_x(kn)

        for k in range(_NITEMS - _NBUF, _NITEMS):
            if k >= 0:
                scat_d[k].wait()

    return sc_add


_sc_add = _make_sc_kernel()


def kernel(x, pos_table):
    b, s, d = x.shape
    out = _sc_add(x.reshape(-1), pos_table[:s].reshape(-1))
    return out.reshape(b, s, d)


# speedup vs baseline: 4.4654x; 4.4654x over previous
import jax
import jax.numpy as jnp
from jax.experimental import pallas as pl

_BS = 1024


def _body(x_ref, pos_ref, o_ref):
    o_ref[...] = x_ref[...] + pos_ref[...][None, :, :]


def kernel(x, pos_table):
    b, s, d = x.shape
    return pl.pallas_call(
        _body,
        grid=(s // _BS, b),
        in_specs=[
            pl.BlockSpec((1, _BS, d), lambda i, j: (j, i, 0)),
            pl.BlockSpec((_BS, d), lambda i, j: (i, 0)),
        ],
        out_specs=pl.BlockSpec((1, _BS, d), lambda i, j: (j, i, 0)),
        out_shape=jax.ShapeDtypeStruct((b, s, d), x.dtype),
    )(x, pos_table)


# speedup vs baseline: 4.7470x; 1.0631x over previous
import jax
import jax.numpy as jnp
from jax.experimental import pallas as pl

_BS = 2048


def _body(x_ref, pos_ref, o_ref):
    o_ref[...] = x_ref[...] + pos_ref[...][None, :, :]


def kernel(x, pos_table):
    b, s, d = x.shape
    return pl.pallas_call(
        _body,
        grid=(s // _BS, b),
        in_specs=[
            pl.BlockSpec((1, _BS, d), lambda i, j: (j, i, 0)),
            pl.BlockSpec((_BS, d), lambda i, j: (i, 0)),
        ],
        out_specs=pl.BlockSpec((1, _BS, d), lambda i, j: (j, i, 0)),
        out_shape=jax.ShapeDtypeStruct((b, s, d), x.dtype),
    )(x, pos_table)
